# baseline (speedup 1.0000x reference)
import functools

import numpy as np

import jax
import jax.numpy as jnp
from jax import lax
from jax.experimental import pallas as pl
from jax.experimental.pallas import tpu as pltpu
from jax.experimental.pallas import tpu_sc as plsc

B = 16
C = 6
L = 4096
CH = 16
NCHUNK = L // CH
VOCAB = 3126
TABLE_PAD = 3200

_E_NP = np.zeros((128, 640), np.float32)
_E_NP[np.arange(128).repeat(5), np.arange(640)] = 1.0


def _tc_pre(samp_ref, w_ref):
    best = samp_ref[0]
    base = jnp.zeros((B, L), jnp.int32)
    for ch in range(1, C):
        s = samp_ref[ch]
        m = s > best
        best = jnp.where(m, s, best)
        base = jnp.where(m, ch, base)
    prev = pltpu.roll(base, 1, axis=1)
    lane = lax.broadcasted_iota(jnp.int32, (B, L), 1)
    prev = jnp.where(lane == 0, 0, prev)
    app = (base != 0) & (base != prev)
    appf = app.astype(jnp.float32)
    ii = lax.broadcasted_iota(jnp.int32, (128, 128), 0)
    jj = lax.broadcasted_iota(jnp.int32, (128, 128), 1)
    t = (ii <= jj).astype(jnp.float32)
    run = jnp.zeros((B, 1), jnp.float32)
    cs = []
    for k in range(L // 128):
        blk = appf[:, 128 * k:128 * (k + 1)]
        intra = jnp.dot(blk, t, preferred_element_type=jnp.float32)
        cs.append(intra + run)
        run = run + intra[:, 127:128]
    c = jnp.concatenate(cs, axis=1).astype(jnp.int32)
    w_ref[...] = c * 16 + jnp.where(app, 8 + base - 1, 0)


def _sc_decode_lookup(w_hbm, table_hbm, emb_hbm, part_hbm, w_v, table_v,
                      d_v, emb_v, part_v):
    wid = lax.axis_index("s") * 2 + lax.axis_index("c")

    @pl.when(wid < B)
    def _():
        pltpu.sync_copy(w_hbm.at[wid], w_v)
        pltpu.sync_copy(table_hbm, table_v)
        d_v[pl.ds(0, CH)] = jnp.zeros((CH,), jnp.int32)

        def chunk(j, carry):
            s0, s1 = carry
            off = j * CH
            w = w_v[pl.ds(off, CH)]
            c = lax.shift_right_logical(w, 4)
            app = (w & 8) != 0
            digit = w & 7
            plsc.store_scatter(d_v, [c], digit, mask=app)
            val = jnp.zeros((CH,), jnp.int32)
            for k, p5 in enumerate((1, 5, 25, 125, 625)):
                g = plsc.load_gather(d_v, [jnp.maximum(c - k, 0)])
                val = val + g * p5
            kmer = jnp.where(c >= 5, val + 1, 0)
            emb = plsc.load_gather(table_v, [kmer])
            emb_v[pl.ds(off, CH)] = emb
            return (s0 + emb, s1 + emb * emb)

        z = jnp.zeros((CH,), jnp.float32)
        s0, s1 = lax.fori_loop(0, NCHUNK, chunk, (z, z), unroll=2)
        part_v[0, :] = s0
        part_v[1, :] = s1
        pltpu.sync_copy(emb_v, emb_hbm.at[wid])
        pltpu.sync_copy(part_v, part_hbm.at[wid])


@functools.partial(
    pl.kernel,
    out_type=(
        jax.ShapeDtypeStruct((B, L), jnp.float32),
        jax.ShapeDtypeStruct((B, 2, CH), jnp.float32),
    ),
    mesh=plsc.VectorSubcoreMesh(core_axis_name="c", subcore_axis_name="s"),
    compiler_params=pltpu.CompilerParams(needs_layout_passes=False),
    scratch_types=[
        pltpu.VMEM((L,), jnp.int32),
        pltpu.VMEM((TABLE_PAD,), jnp.float32),
        pltpu.VMEM((L + 8,), jnp.int32),
        pltpu.VMEM((L,), jnp.float32),
        pltpu.VMEM((2, CH), jnp.float32),
    ],
)
def _sc_kernel(w_hbm, table_hbm, emb_hbm, part_hbm, w_v, table_v, d_v,
               emb_v, part_v):
    _sc_decode_lookup(w_hbm, table_hbm, emb_hbm, part_hbm, w_v, table_v,
                      d_v, emb_v, part_v)


def _tc_finalize(emb_ref, part_ref, w_ref, b_ref, e_ref, o_ref):
    p = part_ref[...]
    n = B * L
    s0 = jnp.sum(p[:, 0, :])
    s1 = jnp.sum(p[:, 1, :])
    mean = s0 / n
    var = s1 / n - mean * mean
    scale = w_ref[0] * lax.rsqrt(var + 1e-5)
    shift = b_ref[0] - mean * scale
    e = e_ref[...]
    for k in range(L // 128):
        yk = emb_ref[:, 128 * k:128 * (k + 1)] * scale + shift
        ok = jnp.dot(yk, e, preferred_element_type=jnp.float32)
        for r in range(5):
            o_ref[:, 5 * k + r, :] = ok[:, 128 * r:128 * (r + 1)]


def kernel(sampling, table, bn_weight, bn_bias):
    samp_t = jnp.transpose(sampling, (1, 0, 2))
    w = pl.pallas_call(
        _tc_pre,
        out_shape=jax.ShapeDtypeStruct((B, L), jnp.int32),
    )(samp_t)
    table_flat = jnp.pad(table[:, 0], (0, TABLE_PAD - VOCAB))
    emb, part = _sc_kernel(w, table_flat)
    out = pl.pallas_call(
        _tc_finalize,
        out_shape=jax.ShapeDtypeStruct((B, 5 * L // 128, 128), jnp.float32),
        in_specs=[
            pl.BlockSpec(memory_space=pltpu.VMEM),
            pl.BlockSpec(memory_space=pltpu.VMEM),
            pl.BlockSpec(memory_space=pltpu.SMEM),
            pl.BlockSpec(memory_space=pltpu.SMEM),
            pl.BlockSpec(memory_space=pltpu.VMEM),
        ],
        out_specs=pl.BlockSpec(memory_space=pltpu.VMEM),
    )(emb, part, bn_weight, bn_bias, jnp.asarray(_E_NP))
    return out.reshape(B, 5 * L, 1)


# speedup vs baseline: 1.0701x; 1.0701x over previous
import functools

import numpy as np

import jax
import jax.numpy as jnp
from jax import lax
from jax.experimental import pallas as pl
from jax.experimental.pallas import tpu as pltpu
from jax.experimental.pallas import tpu_sc as plsc

B = 16
C = 6
L = 4096
CH = 16
NCHUNK = L // CH
VOCAB = 3126
TABLE_PAD = 3200

_E_NP = np.zeros((128, 640), np.float32)
_E_NP[np.arange(128).repeat(5), np.arange(640)] = 1.0


def _tc_pre(samp_ref, w_ref):
    best = samp_ref[0]
    base = jnp.zeros((B, L), jnp.int32)
    for ch in range(1, C):
        s = samp_ref[ch]
        m = s > best
        best = jnp.where(m, s, best)
        base = jnp.where(m, ch, base)
    prev = pltpu.roll(base, 1, axis=1)
    lane = lax.broadcasted_iota(jnp.int32, (B, L), 1)
    prev = jnp.where(lane == 0, 0, prev)
    app = (base != 0) & (base != prev)
    appf = app.astype(jnp.float32)
    ii = lax.broadcasted_iota(jnp.int32, (128, 128), 0)
    jj = lax.broadcasted_iota(jnp.int32, (128, 128), 1)
    t = (ii <= jj).astype(jnp.float32)
    run = jnp.zeros((B, 1), jnp.float32)
    cs = []
    for k in range(L // 128):
        blk = appf[:, 128 * k:128 * (k + 1)]
        intra = jnp.dot(blk, t, preferred_element_type=jnp.float32)
        cs.append(intra + run)
        run = run + intra[:, 127:128]
    c = jnp.concatenate(cs, axis=1).astype(jnp.int32)
    w_ref[...] = (c + 8) * 16 + jnp.where(app, 8 + base - 1, 0)


def _sc_decode_lookup(w_hbm, table_hbm, emb_hbm, w_v, table_v, d_v,
                      emb_v):
    wid = lax.axis_index("s") * 2 + lax.axis_index("c")

    @pl.when(wid < B)
    def _():
        pltpu.sync_copy(w_hbm.at[wid], w_v)
        pltpu.sync_copy(table_hbm, table_v)
        d_v[pl.ds(0, CH)] = jnp.zeros((CH,), jnp.int32)

        def chunk(j, _):
            off = j * CH
            w = w_v[pl.ds(off, CH)]
            c8 = lax.shift_right_logical(w, 4)
            app = (w & 8) != 0
            digit = w & 7
            plsc.store_scatter(d_v, [c8], digit, mask=app)
            val = jnp.zeros((CH,), jnp.int32)
            for k, p5 in enumerate((1, 5, 25, 125, 625)):
                g = plsc.load_gather(d_v, [c8 - k])
                val = val + g * p5
            kmer = jnp.where(c8 >= 13, val + 1, 0)
            emb = plsc.load_gather(table_v, [kmer])
            emb_v[pl.ds(off, CH)] = emb
            return 0

        lax.fori_loop(0, NCHUNK, chunk, 0)
        pltpu.sync_copy(emb_v, emb_hbm.at[wid])


@functools.partial(
    pl.kernel,
    out_type=jax.ShapeDtypeStruct((B, L), jnp.float32),
    mesh=plsc.VectorSubcoreMesh(core_axis_name="c", subcore_axis_name="s"),
    compiler_params=pltpu.CompilerParams(needs_layout_passes=False),
    scratch_types=[
        pltpu.VMEM((L,), jnp.int32),
        pltpu.VMEM((TABLE_PAD,), jnp.float32),
        pltpu.VMEM((L + 24,), jnp.int32),
        pltpu.VMEM((L,), jnp.float32),
    ],
)
def _sc_kernel(w_hbm, table_hbm, emb_hbm, w_v, table_v, d_v, emb_v):
    _sc_decode_lookup(w_hbm, table_hbm, emb_hbm, w_v, table_v, d_v, emb_v)


def _tc_finalize(emb_ref, w_ref, b_ref, e_ref, o_ref):
    x = emb_ref[...]
    n = B * L
    mean = jnp.sum(x) / n
    var = jnp.sum(x * x) / n - mean * mean
    scale = w_ref[0] * lax.rsqrt(var + 1e-5)
    shift = b_ref[0] - mean * scale
    e = e_ref[...]
    for k in range(L // 128):
        yk = emb_ref[:, 128 * k:128 * (k + 1)] * scale + shift
        ok = jnp.dot(yk, e, preferred_element_type=jnp.float32)
        for r in range(5):
            o_ref[:, 5 * k + r, :] = ok[:, 128 * r:128 * (r + 1)]


def kernel(sampling, table, bn_weight, bn_bias):
    samp_t = jnp.transpose(sampling, (1, 0, 2))
    w = pl.pallas_call(
        _tc_pre,
        out_shape=jax.ShapeDtypeStruct((B, L), jnp.int32),
    )(samp_t)
    table_flat = jnp.pad(table[:, 0], (0, TABLE_PAD - VOCAB))
    emb = _sc_kernel(w, table_flat)
    out = pl.pallas_call(
        _tc_finalize,
        out_shape=jax.ShapeDtypeStruct((B, 5 * L // 128, 128), jnp.float32),
        in_specs=[
            pl.BlockSpec(memory_space=pltpu.VMEM),
            pl.BlockSpec(memory_space=pltpu.SMEM),
            pl.BlockSpec(memory_space=pltpu.SMEM),
            pl.BlockSpec(memory_space=pltpu.VMEM),
        ],
        out_specs=pl.BlockSpec(memory_space=pltpu.VMEM),
    )(emb, bn_weight, bn_bias, jnp.asarray(_E_NP))
    return out.reshape(B, 5 * L, 1)
